# baseline (speedup 1.0000x reference)
import functools

import jax
import jax.numpy as jnp
from jax import lax
from jax.experimental import pallas as pl
from jax.experimental.pallas import tpu as pltpu
from jax.experimental.pallas import tpu_sc as plsc

_EMBED_DIM = 128
_D_MODEL = 2 * _EMBED_DIM
_HIDDEN = 2 * _D_MODEL
_STEPS = 4
_SUP_PAD = 8
_SUP_ROWS = 2 * _SUP_PAD
_NSEL = 4 * _D_MODEL

_NC = 2
_NS = 16
_NW = _NC * _NS


def _sc_gather(table, idx1, idx2, supidx):
    n = idx1.shape[0]
    b_per_w = n // _NS
    mesh = plsc.VectorSubcoreMesh(core_axis_name="c", subcore_axis_name="s")

    @functools.partial(
        pl.kernel,
        mesh=mesh,
        out_type=(
            jax.ShapeDtypeStruct((n, _EMBED_DIM), jnp.float32),
            jax.ShapeDtypeStruct((n, _EMBED_DIM), jnp.float32),
            jax.ShapeDtypeStruct((_SUP_ROWS, _EMBED_DIM), jnp.float32),
        ),
        scratch_types=[
            pltpu.VMEM((b_per_w,), jnp.int32),
            pltpu.VMEM((b_per_w, _EMBED_DIM), jnp.float32),
            pltpu.VMEM((_SUP_ROWS,), jnp.int32),
            pltpu.VMEM((_SUP_ROWS, _EMBED_DIM), jnp.float32),
            pltpu.SemaphoreType.DMA,
        ],
    )
    def gather_kernel(table_hbm, i1_hbm, i2_hbm, sf_hbm, o1_hbm, o2_hbm,
                      os_hbm, idx_v, rows_v, idxs_v, rows_s, sem):
        wid = lax.axis_index("s") * _NC + lax.axis_index("c")
        half = wid // _NS
        woff = wid % _NS
        base = woff * b_per_w

        @pl.when(half == 0)
        def _():
            pltpu.sync_copy(i1_hbm.at[pl.ds(base, b_per_w)], idx_v)

        @pl.when(half == 1)
        def _():
            pltpu.sync_copy(i2_hbm.at[pl.ds(base, b_per_w)], idx_v)

        pltpu.async_copy(table_hbm.at[idx_v], rows_v, sem).wait()

        @pl.when(half == 0)
        def _():
            pltpu.sync_copy(rows_v, o1_hbm.at[pl.ds(base, b_per_w)])

        @pl.when(half == 1)
        def _():
            pltpu.sync_copy(rows_v, o2_hbm.at[pl.ds(base, b_per_w)])

        @pl.when(wid == _NW - 1)
        def _():
            pltpu.sync_copy(sf_hbm, idxs_v)
            pltpu.async_copy(table_hbm.at[idxs_v], rows_s, sem).wait()
            pltpu.sync_copy(rows_s, os_hbm)

    return gather_kernel(table, idx1, idx2, supidx)


def _sigmoid_pre(x):
    return 0.5 + 0.5 * jnp.tanh(x)


def _gate_rows(w):
    return jnp.concatenate(
        [w[g * _HIDDEN:g * _HIDDEN + _D_MODEL] for g in range(4)], axis=0)


def _matcher_body(few, nsplit, qa_ref, qb_ref, sp_ref, p1w_ref, p1b_ref,
                  p2w_ref, p2b_ref, lna_ref, lnb_ref, wihs_ref, whhs_ref,
                  bsel_ref, out_ref, sg_scr, sr_scr):
    f32 = jnp.float32
    dims = (((1,), (1,)), ((), ()))
    ed = _EMBED_DIM
    d = _D_MODEL

    @pl.when(pl.program_id(0) == 0)
    def _():
        s = sp_ref[...].reshape(_SUP_PAD, _D_MODEL)
        h1 = lax.dot_general(s, p1w_ref[...], dims,
                             preferred_element_type=f32)
        h1 = jnp.maximum(h1 + p1b_ref[...], 0.0)
        h2 = lax.dot_general(h1, p2w_ref[...], dims,
                             preferred_element_type=f32)
        z = h2 + p2b_ref[...] + s
        mu = jnp.mean(z, axis=1, keepdims=True)
        zc = z - mu
        var = jnp.sum(zc * zc, axis=1, keepdims=True) / (_D_MODEL - 1)
        sgv = zc / (jnp.sqrt(var) + 1e-6) * lna_ref[...] + lnb_ref[...]
        row = lax.broadcasted_iota(jnp.int32, (_SUP_PAD, 1), 0)
        sgv = jnp.where(row < few, sgv, 0.0)
        sg_scr[...] = sgv
        sr_scr[...] = lax.dot_general(sgv, whhs_ref[...][:, d:], dims,
                                      preferred_element_type=f32)

    sg = sg_scr[...]
    s_r = sr_scr[...]
    wihs = wihs_ref[...]
    whh_h = whhs_ref[...][:, :d]

    col = lax.broadcasted_iota(jnp.int32, (1, _SUP_PAD), 1)
    hb = qa_ref.shape[0] // nsplit
    for p in range(nsplit):
        rows = pl.ds(p * hb, hb)
        qa = qa_ref[rows, :]
        qb = qb_ref[rows, :]
        g0 = (lax.dot_general(qa, wihs[:, :ed], dims,
                              preferred_element_type=f32)
              + lax.dot_general(qb, wihs[:, ed:], dims,
                                preferred_element_type=f32)
              + bsel_ref[...])
        g0qh = (g0
                + lax.dot_general(qa, whh_h[:, :ed], dims,
                                  preferred_element_type=f32)
                + lax.dot_general(qb, whh_h[:, ed:], dims,
                                  preferred_element_type=f32))
        qs = (lax.dot_general(qa, sg[:, :ed], dims,
                              preferred_element_type=f32)
              + lax.dot_general(qb, sg[:, ed:], dims,
                                preferred_element_type=f32))

        gates = g0
        c = None
        for t in range(_STEPS):
            gi = _sigmoid_pre(gates[:, :d])
            gg = jnp.tanh(gates[:, 2 * d:3 * d])
            if c is None:
                c = gi * gg
            else:
                gf = _sigmoid_pre(gates[:, d:2 * d])
                c = gf * c + gi * gg
            go = _sigmoid_pre(gates[:, 3 * d:])
            hch = go * jnp.tanh(c)
            logits = qs + lax.dot_general(hch, sg, dims,
                                          preferred_element_type=f32)
            if t == _STEPS - 1:
                out_ref[rows, :] = logits[:, :few]
            else:
                lm = jnp.where(col < few, logits, -1e30)
                m = jnp.max(lm, axis=1, keepdims=True)
                e = jnp.exp(lm - m)
                attn = e / jnp.sum(e, axis=1, keepdims=True)
                gates = (g0qh
                         + lax.dot_general(hch, whh_h, dims,
                                           preferred_element_type=f32)
                         + jnp.dot(attn, s_r, preferred_element_type=f32))


def _matcher_call(q1, q2, srows, proj1_w, proj1_b, proj2_w, proj2_b, ln_a,
                  ln_b, w_ih, w_hh, b_ih, b_hh, few, blk):
    batch = q1.shape[0]
    nb = batch // blk
    gscale = jnp.concatenate(
        [jnp.full((_D_MODEL, 1), 0.5 if g != 2 else 1.0, jnp.float32)
         for g in range(4)], axis=0)
    wihs = _gate_rows(w_ih) * gscale
    whhs = _gate_rows(w_hh) * gscale
    bsel = (_gate_rows((b_ih + b_hh).reshape(-1, 1))
            * gscale).reshape(1, _NSEL)
    whole = lambda shape: pl.BlockSpec(shape, lambda i: (0, 0))
    return pl.pallas_call(
        functools.partial(_matcher_body, few, 2),
        grid=(nb,),
        in_specs=[
            pl.BlockSpec((blk, _EMBED_DIM), lambda i: (i, 0)),
            pl.BlockSpec((blk, _EMBED_DIM), lambda i: (i, 0)),
            whole((_SUP_ROWS, _EMBED_DIM)),
            whole(proj1_w.shape),
            whole((1, proj1_b.shape[0])),
            whole(proj2_w.shape),
            whole((1, proj2_b.shape[0])),
            whole((1, ln_a.shape[0])),
            whole((1, ln_b.shape[0])),
            whole(wihs.shape),
            whole(whhs.shape),
            whole((1, _NSEL)),
        ],
        out_specs=pl.BlockSpec((blk, few), lambda i: (i, 0)),
        out_shape=jax.ShapeDtypeStruct((batch, few), jnp.float32),
        scratch_shapes=[
            pltpu.VMEM((_SUP_PAD, _D_MODEL), jnp.float32),
            pltpu.VMEM((_SUP_PAD, _NSEL), jnp.float32),
        ],
        compiler_params=pltpu.CompilerParams(
            dimension_semantics=("arbitrary",)),
    )(q1, q2, srows, proj1_w, proj1_b.reshape(1, -1), proj2_w,
      proj2_b.reshape(1, -1), ln_a.reshape(1, -1), ln_b.reshape(1, -1),
      wihs, whhs, bsel)


def kernel(query, support, table, proj1_w, proj1_b, proj2_w, proj2_b,
           ln_a, ln_b, w_ih, w_hh, b_ih, b_hh):
    few = support.shape[0]
    zero_row = table.shape[0] - 1
    supidx = jnp.concatenate(
        [support.reshape(-1).astype(jnp.int32),
         jnp.full((_SUP_ROWS - 2 * few,), zero_row, jnp.int32)])
    qi = query.astype(jnp.int32)
    q1, q2, srows = _sc_gather(table, qi[:, 0], qi[:, 1], supidx)
    return _matcher_call(q1, q2, srows, proj1_w, proj1_b, proj2_w, proj2_b,
                         ln_a, ln_b, w_ih, w_hh, b_ih, b_hh, few, blk=2048)


# speedup vs baseline: 1.0359x; 1.0359x over previous
import functools

import jax
import jax.numpy as jnp
from jax import lax
from jax.experimental import pallas as pl
from jax.experimental.pallas import tpu as pltpu
from jax.experimental.pallas import tpu_sc as plsc

_EMBED_DIM = 128
_D_MODEL = 2 * _EMBED_DIM
_HIDDEN = 2 * _D_MODEL
_STEPS = 4
_SUP_PAD = 8
_SUP_ROWS = 2 * _SUP_PAD
_NSEL = 4 * _D_MODEL

_NC = 2
_NS = 16
_NW = _NC * _NS


def _sc_gather(table, idx1, idx2, supidx):
    n = idx1.shape[0]
    b_per_w = n // _NS
    mesh = plsc.VectorSubcoreMesh(core_axis_name="c", subcore_axis_name="s")

    @functools.partial(
        pl.kernel,
        mesh=mesh,
        out_type=(
            jax.ShapeDtypeStruct((n, _EMBED_DIM), jnp.float32),
            jax.ShapeDtypeStruct((n, _EMBED_DIM), jnp.float32),
            jax.ShapeDtypeStruct((_SUP_ROWS, _EMBED_DIM), jnp.float32),
        ),
        scratch_types=[
            pltpu.VMEM((b_per_w,), jnp.int32),
            pltpu.VMEM((b_per_w, _EMBED_DIM), jnp.float32),
            pltpu.VMEM((_SUP_ROWS,), jnp.int32),
            pltpu.VMEM((_SUP_ROWS, _EMBED_DIM), jnp.float32),
            pltpu.SemaphoreType.DMA,
        ],
    )
    def gather_kernel(table_hbm, i1_hbm, i2_hbm, sf_hbm, o1_hbm, o2_hbm,
                      os_hbm, idx_v, rows_v, idxs_v, rows_s, sem):
        wid = lax.axis_index("s") * _NC + lax.axis_index("c")
        half = wid // _NS
        woff = wid % _NS
        base = woff * b_per_w

        @pl.when(half == 0)
        def _():
            pltpu.sync_copy(i1_hbm.at[pl.ds(base, b_per_w)], idx_v)

        @pl.when(half == 1)
        def _():
            pltpu.sync_copy(i2_hbm.at[pl.ds(base, b_per_w)], idx_v)

        pltpu.async_copy(table_hbm.at[idx_v], rows_v, sem).wait()

        @pl.when(half == 0)
        def _():
            pltpu.sync_copy(rows_v, o1_hbm.at[pl.ds(base, b_per_w)])

        @pl.when(half == 1)
        def _():
            pltpu.sync_copy(rows_v, o2_hbm.at[pl.ds(base, b_per_w)])

        @pl.when(wid == _NW - 1)
        def _():
            pltpu.sync_copy(sf_hbm, idxs_v)
            pltpu.async_copy(table_hbm.at[idxs_v], rows_s, sem).wait()
            pltpu.sync_copy(rows_s, os_hbm)

    return gather_kernel(table, idx1, idx2, supidx)


def _sigmoid_pre(x):
    return 0.5 + 0.5 * jnp.tanh(x)


def _gate_rows(w):
    return jnp.concatenate(
        [w[g * _HIDDEN:g * _HIDDEN + _D_MODEL] for g in range(4)], axis=0)


def _matcher_body(few, nsplit, qa_ref, qb_ref, sp_ref, p1w_ref, p1b_ref,
                  p2w_ref, p2b_ref, lna_ref, lnb_ref, wihs_ref, whhs_ref,
                  bsel_ref, out_ref, sg_scr, sr_scr):
    f32 = jnp.float32
    dims = (((1,), (1,)), ((), ()))
    ed = _EMBED_DIM
    d = _D_MODEL

    @pl.when(pl.program_id(0) == 0)
    def _():
        s = sp_ref[...].reshape(_SUP_PAD, _D_MODEL)
        h1 = lax.dot_general(s, p1w_ref[...], dims,
                             preferred_element_type=f32)
        h1 = jnp.maximum(h1 + p1b_ref[...], 0.0)
        h2 = lax.dot_general(h1, p2w_ref[...], dims,
                             preferred_element_type=f32)
        z = h2 + p2b_ref[...] + s
        mu = jnp.mean(z, axis=1, keepdims=True)
        zc = z - mu
        var = jnp.sum(zc * zc, axis=1, keepdims=True) / (_D_MODEL - 1)
        sgv = zc / (jnp.sqrt(var) + 1e-6) * lna_ref[...] + lnb_ref[...]
        row = lax.broadcasted_iota(jnp.int32, (_SUP_PAD, 1), 0)
        sgv = jnp.where(row < few, sgv, 0.0)
        sg_scr[...] = sgv
        sr_scr[...] = lax.dot_general(sgv, whhs_ref[...][:, d:], dims,
                                      preferred_element_type=f32)

    sg = sg_scr[...]
    s_r = sr_scr[...]
    wihs = wihs_ref[...]
    whh_h = whhs_ref[...][:, :d]

    col = lax.broadcasted_iota(jnp.int32, (1, _SUP_PAD), 1)
    hb = qa_ref.shape[0] // nsplit
    for p in range(nsplit):
        rows = pl.ds(p * hb, hb)
        qa = qa_ref[rows, :]
        qb = qb_ref[rows, :]
        g0 = (lax.dot_general(qa, wihs[:, :ed], dims,
                              preferred_element_type=f32)
              + lax.dot_general(qb, wihs[:, ed:], dims,
                                preferred_element_type=f32)
              + bsel_ref[...])
        g0qh = (g0
                + lax.dot_general(qa, whh_h[:, :ed], dims,
                                  preferred_element_type=f32)
                + lax.dot_general(qb, whh_h[:, ed:], dims,
                                  preferred_element_type=f32))
        qs = (lax.dot_general(qa, sg[:, :ed], dims,
                              preferred_element_type=f32)
              + lax.dot_general(qb, sg[:, ed:], dims,
                                preferred_element_type=f32))

        gates = g0
        c = None
        for t in range(_STEPS):
            gi = _sigmoid_pre(gates[:, :d])
            gg = jnp.tanh(gates[:, 2 * d:3 * d])
            if c is None:
                c = gi * gg
            else:
                gf = _sigmoid_pre(gates[:, d:2 * d])
                c = gf * c + gi * gg
            go = _sigmoid_pre(gates[:, 3 * d:])
            hch = go * jnp.tanh(c)
            logits = qs + lax.dot_general(hch, sg, dims,
                                          preferred_element_type=f32)
            if t == _STEPS - 1:
                out_ref[rows, :] = logits[:, :few]
            else:
                lm = jnp.where(col < few, logits, -1e30)
                m = jnp.max(lm, axis=1, keepdims=True)
                e = jnp.exp(lm - m)
                attn = e / jnp.sum(e, axis=1, keepdims=True)
                gates = (g0qh
                         + lax.dot_general(hch, whh_h, dims,
                                           preferred_element_type=f32)
                         + jnp.dot(attn, s_r, preferred_element_type=f32))


def _matcher_call(q1, q2, srows, proj1_w, proj1_b, proj2_w, proj2_b, ln_a,
                  ln_b, w_ih, w_hh, b_ih, b_hh, few, blk):
    batch = q1.shape[0]
    nb = batch // blk
    gscale = jnp.concatenate(
        [jnp.full((_D_MODEL, 1), 0.5 if g != 2 else 1.0, jnp.float32)
         for g in range(4)], axis=0)
    wihs = _gate_rows(w_ih) * gscale
    whhs = _gate_rows(w_hh) * gscale
    bsel = (_gate_rows((b_ih + b_hh).reshape(-1, 1))
            * gscale).reshape(1, _NSEL)
    whole = lambda shape: pl.BlockSpec(shape, lambda i: (0, 0))
    return pl.pallas_call(
        functools.partial(_matcher_body, few, 2),
        grid=(nb,),
        in_specs=[
            pl.BlockSpec((blk, _EMBED_DIM), lambda i: (i, 0)),
            pl.BlockSpec((blk, _EMBED_DIM), lambda i: (i, 0)),
            whole((_SUP_ROWS, _EMBED_DIM)),
            whole(proj1_w.shape),
            whole((1, proj1_b.shape[0])),
            whole(proj2_w.shape),
            whole((1, proj2_b.shape[0])),
            whole((1, ln_a.shape[0])),
            whole((1, ln_b.shape[0])),
            whole(wihs.shape),
            whole(whhs.shape),
            whole((1, _NSEL)),
        ],
        out_specs=pl.BlockSpec((blk, few), lambda i: (i, 0)),
        out_shape=jax.ShapeDtypeStruct((batch, few), jnp.float32),
        scratch_shapes=[
            pltpu.VMEM((_SUP_PAD, _D_MODEL), jnp.float32),
            pltpu.VMEM((_SUP_PAD, _NSEL), jnp.float32),
        ],
        compiler_params=pltpu.CompilerParams(
            dimension_semantics=("arbitrary",)),
    )(q1, q2, srows, proj1_w, proj1_b.reshape(1, -1), proj2_w,
      proj2_b.reshape(1, -1), ln_a.reshape(1, -1), ln_b.reshape(1, -1),
      wihs, whhs, bsel)


def kernel(query, support, table, proj1_w, proj1_b, proj2_w, proj2_b,
           ln_a, ln_b, w_ih, w_hh, b_ih, b_hh):
    few = support.shape[0]
    zero_row = table.shape[0] - 1
    supidx = jnp.concatenate(
        [support.reshape(-1).astype(jnp.int32),
         jnp.full((_SUP_ROWS - 2 * few,), zero_row, jnp.int32)])
    qi = query.astype(jnp.int32)
    q1, q2, srows = _sc_gather(table, qi[:, 0], qi[:, 1], supidx)
    return _matcher_call(q1, q2, srows, proj1_w, proj1_b, proj2_w, proj2_b,
                         ln_a, ln_b, w_ih, w_hh, b_ih, b_hh, few, blk=1024)
